# baseline (speedup 1.0000x reference)
import jax
import jax.numpy as jnp
from jax import lax
from jax.experimental import pallas as pl
from jax.experimental.pallas import tpu as pltpu
from jax.experimental.pallas import tpu_sc as plsc

D = 64
NC, NS = 2, 16
NW = NC * NS
BB = 128
S = 200


def _gather_kernel(idx_hbm, table_hbm, out_hbm, idx_t,
                   rows0, rows1, rows2, rows3, tr0, tr1, tr2, tr3,
                   gs0, gs1, gs2, gs3, ws0, ws1, ws2, ws3):
    wid = lax.axis_index("s") * NC + lax.axis_index("c")
    b0 = wid * BB
    rows = (rows0, rows1, rows2, rows3)
    trs = (tr0, tr1, tr2, tr3)
    g_sem = (gs0, gs1, gs2, gs3)
    w_sem = (ws0, ws1, ws2, ws3)

    pltpu.sync_copy(idx_hbm.at[:, pl.ds(b0, BB)], idx_t)

    lanes = lax.iota(jnp.int32, 16)
    diag = [(lanes + d) & 15 for d in range(16)]
    lcc = [lanes + cc0 for cc0 in range(0, D, 16)]
    rct = [(lanes >> 3) + cc0 // 8 for cc0 in range(0, D, 16)]
    colbase = (lanes & 7) << 7

    def fire_gather(s, b):
        pltpu.async_copy(table_hbm.at[idx_t.at[s]], rows[b], g_sem[b])

    def drain_gather(b):
        pltpu.make_async_copy(table_hbm.at[idx_t.at[0]], rows[b],
                              g_sem[b]).wait()

    def transpose(b):
        @plsc.parallel_loop(0, 16, 1, unroll=4)
        def _t(d):
            diag_d = (lanes + d) & 15
            for bl0 in range(0, BB, 16):
                v1 = diag_d + bl0
                vc = colbase + v1
                for ci in range(D // 16):
                    v = plsc.load_gather(rows[b], [v1, lcc[ci]])
                    plsc.store_scatter(trs[b], [rct[ci], vc], v)

    def fire_wb(s, b):
        pltpu.async_copy(trs[b], out_hbm.at[s, pl.ds(0, 8), wid], w_sem[b])

    def drain_wb(b):
        pltpu.make_async_copy(trs[b], out_hbm.at[0, pl.ds(0, 8), wid],
                              w_sem[b]).wait()

    for b in range(4):
        fire_gather(b, b)

    def quad_body(p, carry):
        for b in range(4):
            s = 4 * p + b
            drain_gather(b)

            @pl.when(p >= 1)
            def _():
                drain_wb(b)

            transpose(b)

            @pl.when(p <= S // 4 - 2)
            def _():
                fire_gather(s + 4, b)

            fire_wb(s, b)
        return carry

    lax.fori_loop(0, S // 4, quad_body, 0, unroll=False)

    for b in range(4):
        drain_wb(b)


NRT = 7812
RT_PER_W = 245


def _detile_kernel(wt_hbm, wtail_hbm, lin_hbm,
                   t0, t1, r0, r1, is0, is1, os0, os1):
    wid = lax.axis_index("s") * NC + lax.axis_index("c")
    tiles = (t0, t1)
    rls = (r0, r1)
    i_sem = (is0, is1)
    o_sem = (os0, os1)
    start = wid * RT_PER_W
    count = jnp.maximum(jnp.minimum(NRT - start, RT_PER_W), 0)

    lanes = lax.iota(jnp.int32, 16)
    diag = [(lanes + d) & 15 for d in range(16)]
    lcc = [lanes + cc0 for cc0 in range(0, D, 16)]

    @pl.when(wid == NW - 1)
    def _():
        pltpu.sync_copy(wtail_hbm, t0.at[pl.ds(0, 32)])
        pltpu.sync_copy(t0.at[pl.ds(0, 32)],
                        lin_hbm.at[pl.ds(NRT * D, 32)])

    def fire_in(i, b):
        rt = start + i
        pltpu.async_copy(wt_hbm.at[:, pl.ds(rt * 128, 128)], tiles[b],
                         i_sem[b])

    def drain_in(b):
        pltpu.make_async_copy(wt_hbm.at[:, pl.ds(0, 128)], tiles[b],
                              i_sem[b]).wait()

    def fire_out(i, b):
        rt = start + i
        pltpu.async_copy(rls[b], lin_hbm.at[pl.ds(rt * D, D)], o_sem[b])

    def drain_out(b):
        pltpu.make_async_copy(rls[b], lin_hbm.at[pl.ds(0, D)],
                              o_sem[b]).wait()

    def transpose(b):
        @plsc.parallel_loop(0, 16, 1, unroll=4)
        def _t(d):
            diag_d = (lanes + d) & 15
            for l0 in range(0, 128, 16):
                vl = diag_d + l0
                dr = vl >> 1
                dc0 = (vl & 1) << 6
                for ci in range(D // 16):
                    v = plsc.load_gather(tiles[b], [lcc[ci], vl])
                    plsc.store_scatter(rls[b], [dr, dc0 + lcc[ci]], v)

    fire_in(0, 0)
    fire_in(1, 1)

    def pair_body(p, carry):
        for b in (0, 1):
            i = 2 * p + b

            @pl.when(i < count)
            def _():
                drain_in(b)

                @pl.when(p >= 1)
                def _():
                    drain_out(b)

                transpose(b)

                @pl.when(i + 2 < count)
                def _():
                    fire_in(i + 2, b)

                fire_out(i, b)
        return carry

    lax.fori_loop(0, (RT_PER_W + 1) // 2, pair_body, 0, unroll=False)
    for b in (0, 1):
        drain_out(b)


def _detile_table(weight):
    V = weight.shape[0]
    wtail = lax.slice(weight, (NRT * 128, 0), (V, D)).reshape(32, 128)
    mesh = plsc.VectorSubcoreMesh(core_axis_name="c", subcore_axis_name="s")
    k = pl.kernel(
        _detile_kernel,
        out_type=jax.ShapeDtypeStruct((V // 2, 128), jnp.float32),
        mesh=mesh,
        scratch_types=[
            pltpu.VMEM((D, 128), jnp.float32),
            pltpu.VMEM((D, 128), jnp.float32),
            pltpu.VMEM((D, 128), jnp.float32),
            pltpu.VMEM((D, 128), jnp.float32),
            pltpu.SemaphoreType.DMA,
            pltpu.SemaphoreType.DMA,
            pltpu.SemaphoreType.DMA,
            pltpu.SemaphoreType.DMA,
        ],
        compiler_params=pltpu.CompilerParams(use_tc_tiling_on_sc=True,
                                             needs_layout_passes=False),
    )
    lin = k(weight.T, wtail)
    return lin.reshape(V, D)


def kernel(inputs, weight):
    idx_t = inputs.T.astype(jnp.int32)
    table = _detile_table(weight)

    mesh = plsc.VectorSubcoreMesh(core_axis_name="c", subcore_axis_name="s")
    k = pl.kernel(
        _gather_kernel,
        out_type=jax.ShapeDtypeStruct((S, 8, NW, 1024), jnp.float32),
        mesh=mesh,
        scratch_types=[
            pltpu.VMEM((S, BB), jnp.int32),
        ] + [pltpu.VMEM((BB, D), jnp.float32)] * 4
          + [pltpu.VMEM((8, 1024), jnp.float32)] * 4
          + [pltpu.SemaphoreType.DMA] * 8,
        compiler_params=pltpu.CompilerParams(use_tc_tiling_on_sc=False,
                                             needs_layout_passes=False),
    )
    out4 = k(idx_t, table)
    out5 = out4.reshape(S, 8, NW, 8, BB)
    return out5.transpose(2, 4, 0, 1, 3).reshape(
        inputs.shape[0], inputs.shape[1], D)


# speedup vs baseline: 1.0386x; 1.0386x over previous
import jax
import jax.numpy as jnp
from jax import lax
from jax.experimental import pallas as pl
from jax.experimental.pallas import tpu as pltpu
from jax.experimental.pallas import tpu_sc as plsc

D = 64
NC, NS = 2, 16
NW = NC * NS
BB = 128
S = 200


def _gather_kernel(idx_hbm, table_hbm, out_hbm,
                   idx_t, rows0, rows1, tr0, tr1, gs0, gs1, ws0, ws1):
    wid = lax.axis_index("s") * NC + lax.axis_index("c")
    b0 = wid * BB
    rows = (rows0, rows1)
    trs = (tr0, tr1)
    g_sem = (gs0, gs1)
    w_sem = (ws0, ws1)

    pltpu.sync_copy(idx_hbm.at[:, pl.ds(b0, BB)], idx_t)

    lanes = lax.iota(jnp.int32, 16)
    diag = [(lanes + d) & 15 for d in range(16)]
    lcc = [lanes + cc0 for cc0 in range(0, D, 16)]
    rct = [(lanes >> 3) + cc0 // 8 for cc0 in range(0, D, 16)]
    colbase = (lanes & 7) << 7

    def fire_gather(s, b):
        pltpu.async_copy(table_hbm.at[idx_t.at[s]], rows[b], g_sem[b])

    def drain_gather(b):
        pltpu.make_async_copy(table_hbm.at[idx_t.at[0]], rows[b],
                              g_sem[b]).wait()

    def transpose(b):
        @plsc.parallel_loop(0, 16, 1, unroll=4)
        def _t(d):
            diag_d = (lanes + d) & 15
            for bl0 in range(0, BB, 16):
                v1 = diag_d + bl0
                vc = colbase + v1
                for ci in range(D // 16):
                    v = plsc.load_gather(rows[b], [v1, lcc[ci]])
                    plsc.store_scatter(trs[b], [rct[ci], vc], v)

    def fire_wb(s, b):
        pltpu.async_copy(trs[b], out_hbm.at[s, pl.ds(0, 8), wid], w_sem[b])

    def drain_wb(b):
        pltpu.make_async_copy(trs[b], out_hbm.at[0, pl.ds(0, 8), wid],
                              w_sem[b]).wait()

    fire_gather(0, 0)
    fire_gather(1, 1)

    def pair_body(p, carry):
        for b in (0, 1):
            s = 2 * p + b
            drain_gather(b)

            @pl.when(p >= 1)
            def _():
                drain_wb(b)

            transpose(b)

            @pl.when(p <= S // 2 - 2)
            def _():
                fire_gather(s + 2, b)

            fire_wb(s, b)
        return carry

    lax.fori_loop(0, S // 2, pair_body, 0, unroll=False)

    for b in (0, 1):
        drain_wb(b)


NRT = 7812
RT_PER_W = 245


def _detile_kernel(wt_hbm, wtail_hbm, lin_hbm,
                   t0, t1, r0, r1, is0, is1, os0, os1):
    wid = lax.axis_index("s") * NC + lax.axis_index("c")
    tiles = (t0, t1)
    rls = (r0, r1)
    i_sem = (is0, is1)
    o_sem = (os0, os1)
    start = wid * RT_PER_W
    count = jnp.maximum(jnp.minimum(NRT - start, RT_PER_W), 0)

    lanes = lax.iota(jnp.int32, 16)
    diag = [(lanes + d) & 15 for d in range(16)]
    lcc = [lanes + cc0 for cc0 in range(0, D, 16)]

    @pl.when(wid == NW - 1)
    def _():
        pltpu.sync_copy(wtail_hbm, t0.at[pl.ds(0, 32)])
        pltpu.sync_copy(t0.at[pl.ds(0, 32)],
                        lin_hbm.at[pl.ds(NRT * D, 32)])

    def fire_in(i, b):
        rt = start + i
        pltpu.async_copy(wt_hbm.at[:, pl.ds(rt * 128, 128)], tiles[b],
                         i_sem[b])

    def drain_in(b):
        pltpu.make_async_copy(wt_hbm.at[:, pl.ds(0, 128)], tiles[b],
                              i_sem[b]).wait()

    def fire_out(i, b):
        rt = start + i
        pltpu.async_copy(rls[b], lin_hbm.at[pl.ds(rt * D, D)], o_sem[b])

    def drain_out(b):
        pltpu.make_async_copy(rls[b], lin_hbm.at[pl.ds(0, D)],
                              o_sem[b]).wait()

    def transpose(b):
        @plsc.parallel_loop(0, 16, 1, unroll=4)
        def _t(d):
            diag_d = (lanes + d) & 15
            for l0 in range(0, 128, 16):
                vl = diag_d + l0
                dr = vl >> 1
                dc0 = (vl & 1) << 6
                for ci in range(D // 16):
                    v = plsc.load_gather(tiles[b], [lcc[ci], vl])
                    plsc.store_scatter(rls[b], [dr, dc0 + lcc[ci]], v)

    fire_in(0, 0)
    fire_in(1, 1)

    def pair_body(p, carry):
        for b in (0, 1):
            i = 2 * p + b

            @pl.when(i < count)
            def _():
                drain_in(b)

                @pl.when(p >= 1)
                def _():
                    drain_out(b)

                transpose(b)

                @pl.when(i + 2 < count)
                def _():
                    fire_in(i + 2, b)

                fire_out(i, b)
        return carry

    lax.fori_loop(0, (RT_PER_W + 1) // 2, pair_body, 0, unroll=False)
    for b in (0, 1):
        drain_out(b)


def _detile_table(weight):
    V = weight.shape[0]
    wtail = lax.slice(weight, (NRT * 128, 0), (V, D)).reshape(32, 128)
    mesh = plsc.VectorSubcoreMesh(core_axis_name="c", subcore_axis_name="s")
    k = pl.kernel(
        _detile_kernel,
        out_type=jax.ShapeDtypeStruct((V // 2, 128), jnp.float32),
        mesh=mesh,
        scratch_types=[
            pltpu.VMEM((D, 128), jnp.float32),
            pltpu.VMEM((D, 128), jnp.float32),
            pltpu.VMEM((D, 128), jnp.float32),
            pltpu.VMEM((D, 128), jnp.float32),
            pltpu.SemaphoreType.DMA,
            pltpu.SemaphoreType.DMA,
            pltpu.SemaphoreType.DMA,
            pltpu.SemaphoreType.DMA,
        ],
        compiler_params=pltpu.CompilerParams(use_tc_tiling_on_sc=True,
                                             needs_layout_passes=False),
    )
    lin = k(weight.T, wtail)
    return lin.reshape(V, D)


def kernel(inputs, weight):
    idx_t = inputs.T.astype(jnp.int32)
    table = _detile_table(weight)

    mesh = plsc.VectorSubcoreMesh(core_axis_name="c", subcore_axis_name="s")
    k = pl.kernel(
        _gather_kernel,
        out_type=jax.ShapeDtypeStruct((S, 8, NW, 1024), jnp.float32),
        mesh=mesh,
        scratch_types=[
            pltpu.VMEM((S, BB), jnp.int32),
            pltpu.VMEM((BB, D), jnp.float32),
            pltpu.VMEM((BB, D), jnp.float32),
            pltpu.VMEM((8, 1024), jnp.float32),
            pltpu.VMEM((8, 1024), jnp.float32),
            pltpu.SemaphoreType.DMA,
            pltpu.SemaphoreType.DMA,
            pltpu.SemaphoreType.DMA,
            pltpu.SemaphoreType.DMA,
        ],
        compiler_params=pltpu.CompilerParams(use_tc_tiling_on_sc=False,
                                             needs_layout_passes=False),
    )
    out4 = k(idx_t, table)
    out5 = out4.reshape(S, 8, NW, 8, BB)
    return out5.transpose(2, 4, 0, 1, 3).reshape(
        inputs.shape[0], inputs.shape[1], D)


# speedup vs baseline: 1.3448x; 1.2947x over previous
import jax
import jax.numpy as jnp
from jax import lax
from jax.experimental import pallas as pl
from jax.experimental.pallas import tpu as pltpu
from jax.experimental.pallas import tpu_sc as plsc

D = 64
NC, NS = 2, 16
NW = NC * NS
BB = 128
S = 200


def _gather_kernel(idx_hbm, table_hbm, out_hbm,
                   idx_t, rows0, rows1, tr0, tr1, gs0, gs1, ws0, ws1):
    wid = lax.axis_index("s") * NC + lax.axis_index("c")
    b0 = wid * BB
    rows = (rows0, rows1)
    trs = (tr0, tr1)
    g_sem = (gs0, gs1)
    w_sem = (ws0, ws1)

    pltpu.sync_copy(idx_hbm.at[:, pl.ds(b0, BB)], idx_t)

    lanes = lax.iota(jnp.int32, 16)
    diag = [(lanes + d) & 15 for d in range(16)]
    lcc = [lanes + cc0 for cc0 in range(0, D, 16)]
    rct = [(lanes >> 3) + cc0 // 8 for cc0 in range(0, D, 16)]
    colbase = (lanes & 7) << 7

    def fire_gather(s, b):
        pltpu.async_copy(table_hbm.at[idx_t.at[s]], rows[b], g_sem[b])

    def drain_gather(b):
        pltpu.make_async_copy(table_hbm.at[idx_t.at[0]], rows[b],
                              g_sem[b]).wait()

    def transpose(b):
        @plsc.parallel_loop(0, 16, 1, unroll=8)
        def _t(d):
            diag_d = (lanes + d) & 15
            for bl0 in range(0, BB, 16):
                v1 = diag_d + bl0
                vc = colbase + v1
                for ci in range(D // 16):
                    v = plsc.load_gather(rows[b], [v1, lcc[ci]])
                    plsc.store_scatter(trs[b], [rct[ci], vc], v)

    def fire_wb(s, b):
        pltpu.async_copy(trs[b], out_hbm.at[s, pl.ds(0, 8), wid], w_sem[b])

    def drain_wb(b):
        pltpu.make_async_copy(trs[b], out_hbm.at[0, pl.ds(0, 8), wid],
                              w_sem[b]).wait()

    fire_gather(0, 0)
    fire_gather(1, 1)

    def pair_body(p, carry):
        for b in (0, 1):
            s = 2 * p + b
            drain_gather(b)

            @pl.when(p >= 1)
            def _():
                drain_wb(b)

            transpose(b)

            @pl.when(p <= S // 2 - 2)
            def _():
                fire_gather(s + 2, b)

            fire_wb(s, b)
        return carry

    lax.fori_loop(0, S // 2, pair_body, 0, unroll=False)

    for b in (0, 1):
        drain_wb(b)


NRT = 7812
RT_PER_W = 245


def _detile_kernel(wt_hbm, wtail_hbm, lin_hbm,
                   t0, t1, r0, r1, is0, is1, os0, os1):
    wid = lax.axis_index("s") * NC + lax.axis_index("c")
    tiles = (t0, t1)
    rls = (r0, r1)
    i_sem = (is0, is1)
    o_sem = (os0, os1)
    start = wid * RT_PER_W
    count = jnp.maximum(jnp.minimum(NRT - start, RT_PER_W), 0)

    lanes = lax.iota(jnp.int32, 16)
    diag = [(lanes + d) & 15 for d in range(16)]
    lcc = [lanes + cc0 for cc0 in range(0, D, 16)]

    @pl.when(wid == NW - 1)
    def _():
        pltpu.sync_copy(wtail_hbm, t0.at[pl.ds(0, 32)])
        pltpu.sync_copy(t0.at[pl.ds(0, 32)],
                        lin_hbm.at[pl.ds(NRT * D, 32)])

    def fire_in(i, b):
        rt = start + i
        pltpu.async_copy(wt_hbm.at[:, pl.ds(rt * 128, 128)], tiles[b],
                         i_sem[b])

    def drain_in(b):
        pltpu.make_async_copy(wt_hbm.at[:, pl.ds(0, 128)], tiles[b],
                              i_sem[b]).wait()

    def fire_out(i, b):
        rt = start + i
        pltpu.async_copy(rls[b], lin_hbm.at[pl.ds(rt * D, D)], o_sem[b])

    def drain_out(b):
        pltpu.make_async_copy(rls[b], lin_hbm.at[pl.ds(0, D)],
                              o_sem[b]).wait()

    def transpose(b):
        @plsc.parallel_loop(0, 16, 1, unroll=8)
        def _t(d):
            diag_d = (lanes + d) & 15
            for l0 in range(0, 128, 16):
                vl = diag_d + l0
                dr = vl >> 1
                dc0 = (vl & 1) << 6
                for ci in range(D // 16):
                    v = plsc.load_gather(tiles[b], [lcc[ci], vl])
                    plsc.store_scatter(rls[b], [dr, dc0 + lcc[ci]], v)

    fire_in(0, 0)
    fire_in(1, 1)

    def pair_body(p, carry):
        for b in (0, 1):
            i = 2 * p + b

            @pl.when(i < count)
            def _():
                drain_in(b)

                @pl.when(p >= 1)
                def _():
                    drain_out(b)

                transpose(b)

                @pl.when(i + 2 < count)
                def _():
                    fire_in(i + 2, b)

                fire_out(i, b)
        return carry

    lax.fori_loop(0, (RT_PER_W + 1) // 2, pair_body, 0, unroll=False)
    for b in (0, 1):
        drain_out(b)


def _detile_table(weight):
    V = weight.shape[0]
    wtail = lax.slice(weight, (NRT * 128, 0), (V, D)).reshape(32, 128)
    mesh = plsc.VectorSubcoreMesh(core_axis_name="c", subcore_axis_name="s")
    k = pl.kernel(
        _detile_kernel,
        out_type=jax.ShapeDtypeStruct((V // 2, 128), jnp.float32),
        mesh=mesh,
        scratch_types=[
            pltpu.VMEM((D, 128), jnp.float32),
            pltpu.VMEM((D, 128), jnp.float32),
            pltpu.VMEM((D, 128), jnp.float32),
            pltpu.VMEM((D, 128), jnp.float32),
            pltpu.SemaphoreType.DMA,
            pltpu.SemaphoreType.DMA,
            pltpu.SemaphoreType.DMA,
            pltpu.SemaphoreType.DMA,
        ],
        compiler_params=pltpu.CompilerParams(use_tc_tiling_on_sc=True,
                                             needs_layout_passes=False),
    )
    lin = k(weight.T, wtail)
    return lin.reshape(V, D)


def kernel(inputs, weight):
    idx_t = inputs.T.astype(jnp.int32)
    table = _detile_table(weight)

    mesh = plsc.VectorSubcoreMesh(core_axis_name="c", subcore_axis_name="s")
    k = pl.kernel(
        _gather_kernel,
        out_type=jax.ShapeDtypeStruct((S, 8, NW, 1024), jnp.float32),
        mesh=mesh,
        scratch_types=[
            pltpu.VMEM((S, BB), jnp.int32),
            pltpu.VMEM((BB, D), jnp.float32),
            pltpu.VMEM((BB, D), jnp.float32),
            pltpu.VMEM((8, 1024), jnp.float32),
            pltpu.VMEM((8, 1024), jnp.float32),
            pltpu.SemaphoreType.DMA,
            pltpu.SemaphoreType.DMA,
            pltpu.SemaphoreType.DMA,
            pltpu.SemaphoreType.DMA,
        ],
        compiler_params=pltpu.CompilerParams(use_tc_tiling_on_sc=False,
                                             needs_layout_passes=False),
    )
    out4 = k(idx_t, table)
    out5 = out4.reshape(S, 8, NW, 8, BB)
    return out5.transpose(2, 4, 0, 1, 3).reshape(
        inputs.shape[0], inputs.shape[1], D)
